# baseline (speedup 1.0000x reference)
import functools

import jax
import jax.numpy as jnp
from jax import lax
from jax.experimental import pallas as pl
from jax.experimental.pallas import tpu as pltpu
from jax.experimental.pallas import tpu_sc as plsc

_N = 10000
_E = 320000
_D = 128
_H = 128
_C = 40

_NW = 32
_CHUNK = 512
_NCHUNK = _E // _CHUNK
_BASE = _NCHUNK // _NW
_XTRA = _NCHUNK - _NW * _BASE
_CPW = _BASE + 1
_N_PAD = 10112
_NF = _N * _C // 128
_NPF = _N_PAD * _C // 128
_RPT = _N_PAD // 16
_NB = 4
_NS = 2


def _proj_body(feat_ref, norm_ref, w1_ref, w2_ref, b1_ref, out_ref, b12_ref):
    w12 = jnp.dot(w1_ref[...], w2_ref[...], preferred_element_type=jnp.float32)
    h = feat_ref[...] * norm_ref[...]
    out_ref[...] = jnp.dot(h, w12, preferred_element_type=jnp.float32)
    b12_ref[...] = jnp.dot(b1_ref[...], w2_ref[...], preferred_element_type=jnp.float32)


def _mid_body(p_ref, nn_ref, nb_ref, out_ref):
    out_ref[...] = (p_ref[0] + p_ref[1]) * nn_ref[...] + nb_ref[...]


def _fin_body(p_ref, nf_ref, b2_ref, out_ref):
    a = p_ref[0, :_NF, :] + p_ref[1, :_NF, :]
    out_ref[...] = a * nf_ref[...] + b2_ref[...]


def _seg_body(table, eidx, zeros, out, src_v, dst_v, rows_v, acc, sem, sem2):
    c = lax.axis_index("c")
    s = lax.axis_index("s")
    wid = s * 2 + c
    r0 = s * _RPT
    start = wid * _BASE + lax.max(wid - (_NW - _XTRA), 0)
    cnt = _BASE + jnp.where(wid >= _NW - _XTRA, 1, 0)
    pltpu.sync_copy(zeros.at[pl.ds(r0, _RPT)], acc.at[pl.ds(r0, _RPT)])
    pltpu.sync_copy(eidx.at[0, pl.ds(start, _CPW)], src_v)
    pltpu.sync_copy(eidx.at[1, pl.ds(start, _CPW)], dst_v)
    plsc.subcore_barrier()

    for b in range(_NB - _NS):
        pltpu.async_copy(table.at[src_v.at[b]], rows_v.at[b], sem)

    def body(j, carry):
        b = lax.rem(j, _NB)
        pltpu.make_async_copy(table.at[src_v.at[j]], rows_v.at[b], sem).wait()
        pltpu.async_copy(rows_v.at[b], acc.at[dst_v.at[j]], sem2, add=True)

        @pl.when(j >= _NS)
        def _():
            pltpu.make_async_copy(rows_v.at[b], acc.at[dst_v.at[j]], sem2).wait()

        @pl.when(j + _NB - _NS < cnt)
        def _():
            g = j + _NB - _NS
            pltpu.async_copy(table.at[src_v.at[g]], rows_v.at[lax.rem(g, _NB)], sem)

        return carry

    lax.fori_loop(0, cnt, body, 0)
    for _ in range(_NS):
        pltpu.make_async_copy(rows_v.at[0], acc.at[dst_v.at[0]], sem2).wait()
    plsc.subcore_barrier()
    pltpu.sync_copy(acc.at[pl.ds(r0, _RPT)], out.at[c, pl.ds(r0, _RPT)])


_seg_call = pl.kernel(
    _seg_body,
    out_type=jax.ShapeDtypeStruct((2, _N_PAD, _C), jnp.float32),
    mesh=plsc.VectorSubcoreMesh(core_axis_name="c", subcore_axis_name="s"),
    scratch_types=[
        pltpu.VMEM((_CPW, _CHUNK), jnp.int32),
        pltpu.VMEM((_CPW, _CHUNK), jnp.int32),
        pltpu.VMEM((_NB, _CHUNK, _C), jnp.float32),
        pltpu.VMEM_SHARED((_N_PAD, _C), jnp.float32),
        pltpu.SemaphoreType.DMA,
        pltpu.SemaphoreType.DMA,
    ],
    compiler_params=pltpu.CompilerParams(use_tc_tiling_on_sc=False),
)


def kernel(features, norm, edge_index, W1, b1, W2, b2):
    eidx3 = edge_index.reshape(2, _NCHUNK, _CHUNK)
    zeros = jnp.zeros((_N_PAD, _C), jnp.float32)
    b1r = b1.reshape(1, _H)

    f, b12 = pl.pallas_call(
        _proj_body,
        out_shape=[
            jax.ShapeDtypeStruct((_N, _C), jnp.float32),
            jax.ShapeDtypeStruct((8, _C), jnp.float32),
        ],
    )(features, norm, W1, W2, jnp.broadcast_to(b1r, (8, _H)))

    p1 = _seg_call(f, eidx3, zeros)

    norm_p = jnp.pad(norm, ((0, _N_PAD - _N), (0, 0)))
    nn_flat = jnp.broadcast_to(norm_p * norm_p, (_N_PAD, _C)).reshape(_NPF, 128)
    nb_flat = (norm_p * b12[0:1, :]).reshape(_NPF, 128)
    nf_flat = jnp.broadcast_to(norm, (_N, _C)).reshape(_NF, 128)
    b2_flat = jnp.broadcast_to(b2.reshape(1, _C), (_N, _C)).reshape(_NF, 128)

    g_flat = pl.pallas_call(
        _mid_body,
        out_shape=jax.ShapeDtypeStruct((_NPF, 128), jnp.float32),
    )(p1.reshape(2, _NPF, 128), nn_flat, nb_flat)

    p2 = _seg_call(g_flat.reshape(_N_PAD, _C), eidx3, zeros)

    out_flat = pl.pallas_call(
        _fin_body,
        out_shape=jax.ShapeDtypeStruct((_NF, 128), jnp.float32),
    )(p2.reshape(2, _NPF, 128), nf_flat, b2_flat)

    return out_flat.reshape(_N, _C)


# speedup vs baseline: 1.0188x; 1.0188x over previous
import functools

import jax
import jax.numpy as jnp
from jax import lax
from jax.experimental import pallas as pl
from jax.experimental.pallas import tpu as pltpu
from jax.experimental.pallas import tpu_sc as plsc

_N = 10000
_E = 320000
_D = 128
_H = 128
_C = 40

_NW = 32
_CHUNK = 256
_NCHUNK = _E // _CHUNK
_BASE = _NCHUNK // _NW
_XTRA = _NCHUNK - _NW * _BASE
_CPW = _BASE + 1
_N_PAD = 10112
_NF = _N * _C // 128
_NPF = _N_PAD * _C // 128
_RPT = _N_PAD // 16
_NB = 6
_NS = 2


def _proj_body(feat_ref, norm_ref, w1_ref, w2_ref, b1_ref, out_ref, b12_ref):
    w12 = jnp.dot(w1_ref[...], w2_ref[...], preferred_element_type=jnp.float32)
    h = feat_ref[...] * norm_ref[...]
    out_ref[...] = jnp.dot(h, w12, preferred_element_type=jnp.float32)
    b12_ref[...] = jnp.dot(b1_ref[...], w2_ref[...], preferred_element_type=jnp.float32)


def _mid_body(p_ref, nn_ref, nb_ref, out_ref):
    out_ref[...] = (p_ref[0] + p_ref[1]) * nn_ref[...] + nb_ref[...]


def _fin_body(p_ref, nf_ref, b2_ref, out_ref):
    a = p_ref[0, :_NF, :] + p_ref[1, :_NF, :]
    out_ref[...] = a * nf_ref[...] + b2_ref[...]


def _seg_body(table, eidx, zeros, out, src_v, dst_v, rows_v, acc, sem, sem2):
    c = lax.axis_index("c")
    s = lax.axis_index("s")
    wid = s * 2 + c
    r0 = s * _RPT
    start = wid * _BASE + lax.max(wid - (_NW - _XTRA), 0)
    cnt = _BASE + jnp.where(wid >= _NW - _XTRA, 1, 0)
    pltpu.sync_copy(zeros.at[pl.ds(r0, _RPT)], acc.at[pl.ds(r0, _RPT)])
    pltpu.sync_copy(eidx.at[0, pl.ds(start, _CPW)], src_v)
    pltpu.sync_copy(eidx.at[1, pl.ds(start, _CPW)], dst_v)
    plsc.subcore_barrier()

    for b in range(_NB - _NS):
        pltpu.async_copy(table.at[src_v.at[b]], rows_v.at[b], sem)

    def body(j, carry):
        b = lax.rem(j, _NB)
        pltpu.make_async_copy(table.at[src_v.at[j]], rows_v.at[b], sem).wait()
        pltpu.async_copy(rows_v.at[b], acc.at[dst_v.at[j]], sem2, add=True)

        @pl.when(j >= _NS)
        def _():
            pltpu.make_async_copy(rows_v.at[b], acc.at[dst_v.at[j]], sem2).wait()

        @pl.when(j + _NB - _NS < cnt)
        def _():
            g = j + _NB - _NS
            pltpu.async_copy(table.at[src_v.at[g]], rows_v.at[lax.rem(g, _NB)], sem)

        return carry

    lax.fori_loop(0, cnt, body, 0)
    for _ in range(_NS):
        pltpu.make_async_copy(rows_v.at[0], acc.at[dst_v.at[0]], sem2).wait()
    plsc.subcore_barrier()
    pltpu.sync_copy(acc.at[pl.ds(r0, _RPT)], out.at[c, pl.ds(r0, _RPT)])


_seg_call = pl.kernel(
    _seg_body,
    out_type=jax.ShapeDtypeStruct((2, _N_PAD, _C), jnp.float32),
    mesh=plsc.VectorSubcoreMesh(core_axis_name="c", subcore_axis_name="s"),
    scratch_types=[
        pltpu.VMEM((_CPW, _CHUNK), jnp.int32),
        pltpu.VMEM((_CPW, _CHUNK), jnp.int32),
        pltpu.VMEM((_NB, _CHUNK, _C), jnp.float32),
        pltpu.VMEM_SHARED((_N_PAD, _C), jnp.float32),
        pltpu.SemaphoreType.DMA,
        pltpu.SemaphoreType.DMA,
    ],
    compiler_params=pltpu.CompilerParams(use_tc_tiling_on_sc=False),
)


def kernel(features, norm, edge_index, W1, b1, W2, b2):
    eidx3 = edge_index.reshape(2, _NCHUNK, _CHUNK)
    zeros = jnp.zeros((_N_PAD, _C), jnp.float32)
    b1r = b1.reshape(1, _H)

    f, b12 = pl.pallas_call(
        _proj_body,
        out_shape=[
            jax.ShapeDtypeStruct((_N, _C), jnp.float32),
            jax.ShapeDtypeStruct((8, _C), jnp.float32),
        ],
    )(features, norm, W1, W2, jnp.broadcast_to(b1r, (8, _H)))

    p1 = _seg_call(f, eidx3, zeros)

    norm_p = jnp.pad(norm, ((0, _N_PAD - _N), (0, 0)))
    nn_flat = jnp.broadcast_to(norm_p * norm_p, (_N_PAD, _C)).reshape(_NPF, 128)
    nb_flat = (norm_p * b12[0:1, :]).reshape(_NPF, 128)
    nf_flat = jnp.broadcast_to(norm, (_N, _C)).reshape(_NF, 128)
    b2_flat = jnp.broadcast_to(b2.reshape(1, _C), (_N, _C)).reshape(_NF, 128)

    g_flat = pl.pallas_call(
        _mid_body,
        out_shape=jax.ShapeDtypeStruct((_NPF, 128), jnp.float32),
    )(p1.reshape(2, _NPF, 128), nn_flat, nb_flat)

    p2 = _seg_call(g_flat.reshape(_N_PAD, _C), eidx3, zeros)

    out_flat = pl.pallas_call(
        _fin_body,
        out_shape=jax.ShapeDtypeStruct((_NF, 128), jnp.float32),
    )(p2.reshape(2, _NPF, 128), nf_flat, b2_flat)

    return out_flat.reshape(_N, _C)
